# Initial kernel scaffold; baseline (speedup 1.0000x reference)
import jax
import jax.numpy as jnp
from jax.experimental import pallas as pl


def kernel(x1, edge_index1, x2, edge_index2, batch_idx, label1, label2, equ_lamb, params):
    raise NotImplementedError("write your pallas kernel here")


# speedup vs baseline: 1.0609x; 1.0609x over previous
import jax
import jax.numpy as jnp
from jax.experimental import pallas as pl

N = 10000; E = 320000; G = 128; IN = 128; D = 300; H = 600; C = 10
GAMMA = 0.4; INV_LAMB = 0.5


def _matmul_bias_kernel(x_ref, w_ref, b_ref, o_ref):
    o_ref[...] = jnp.dot(x_ref[...], w_ref[...],
                         preferred_element_type=jnp.float32) + b_ref[...]


def _pallas_linear(x, w, b):
    m, k = x.shape
    k2, n = w.shape
    return pl.pallas_call(
        _matmul_bias_kernel,
        out_shape=jax.ShapeDtypeStruct((m, n), jnp.float32),
    )(x, w, b[None, :])


def _layer(x, W, b, src, dst, n, node_mask=None, edge_mask=None):
    if node_mask is not None:
        x = x * node_mask
    m = _pallas_linear(x, W, b)
    msg = m[src]
    if edge_mask is not None:
        msg = msg * edge_mask
    agg = jax.ops.segment_sum(msg, dst, num_segments=n)
    deg = jax.ops.segment_sum(jnp.ones((src.shape[0],), m.dtype), dst, num_segments=n)
    agg = agg / jnp.clip(deg, 1.0)[:, None]
    return jax.nn.relu(agg + m)


def _encoder(x, ei, layers, node_mask=None, edge_mask=None):
    src, dst = ei[0], ei[1]
    h = x
    for (W, b) in layers:
        h = _layer(h, W, b, src, dst, x.shape[0], node_mask, edge_mask)
    return h


def _causaler(x, ei, batch, p):
    enc = _encoder(x, ei, p['enc'])
    node_key = jax.nn.sigmoid(enc @ p['wn'] + p['bn'])
    src, dst = ei[0], ei[1]
    ef = jnp.concatenate([enc[src], enc[dst]], axis=1)
    edge_key = jax.nn.sigmoid(ef @ p['we'] + p['be'])
    nk = jax.ops.segment_sum(node_key[:, 0], batch, num_segments=G)
    nc = jax.ops.segment_sum(jnp.ones((x.shape[0],), x.dtype), batch, num_segments=G)
    eb = batch[src]
    ek = jax.ops.segment_sum(edge_key[:, 0], eb, num_segments=G)
    ec = jax.ops.segment_sum(jnp.ones((src.shape[0],), x.dtype), eb, num_segments=G)
    return node_key, edge_key, nk, nc - nk, ek, ec - ek, jnp.ones((G,), x.dtype), jnp.ones((G,), x.dtype)


def _pool(h, batch):
    s = jax.ops.segment_sum(h, batch, num_segments=G)
    c = jax.ops.segment_sum(jnp.ones((h.shape[0],), h.dtype), batch, num_segments=G)
    return s / jnp.clip(c, 1.0)[:, None]


def _predictor(h, p):
    z = _pallas_linear(h, p['w1'], p['b1'])
    mu = z.mean(0); var = z.var(0)
    z = (z - mu) / jnp.sqrt(var + 1e-5) * p['g'] + p['bt']
    z = jax.nn.relu(z)
    return _pallas_linear(z, p['w2'], p['b2'])


def _reg(cau, env, ratio):
    r = cau / (cau + env + 1e-8)
    return (jnp.abs(r - GAMMA) * ratio).mean()


def kernel(x1, edge_index1, x2, edge_index2, batch_idx, label1, label2, equ_lamb, params):
    xe1 = _encoder(x1, edge_index1, params['front'])
    xe2 = _encoder(x2, edge_index2, params['front'])
    nk1, ek1, nkn1, nen1, ekn1, een1, rn1, re1 = _causaler(x1, edge_index1, batch_idx, params['cau'])
    nk2, ek2, nkn2, nen2, ekn2, een2, rn2, re2 = _causaler(x2, edge_index2, batch_idx, params['cau'])
    h1 = _encoder(xe1, edge_index1, params['back'], node_mask=nk1, edge_mask=ek1)
    h2 = _encoder(xe2, edge_index2, params['back'], node_mask=nk2, edge_mask=ek2)
    lam_n = equ_lamb[batch_idx][:, None]
    x_equ = lam_n * xe1 + (1.0 - lam_n) * xe2
    x_inv = INV_LAMB * xe1 + (1.0 - INV_LAMB) * xe2
    ei_mix = jnp.concatenate([edge_index1, edge_index2], axis=1)
    l_e1 = equ_lamb[batch_idx[edge_index1[0]]]
    l_e2 = 1.0 - equ_lamb[batch_idx[edge_index2[0]]]
    w_equ = jnp.concatenate([l_e1, l_e2])[:, None]
    w_inv = jnp.full((ei_mix.shape[1], 1), INV_LAMB, x1.dtype)
    h_equ = _encoder(x_equ, ei_mix, params['back'], edge_mask=w_equ)
    h_inv = _encoder(x_inv, ei_mix, params['back'], edge_mask=w_inv)
    g1 = _pool(h1, batch_idx); g2 = _pool(h2, batch_idx)
    g_equ = _pool(h_equ, batch_idx); g_inv = _pool(h_inv, batch_idx)
    pred_cau = _predictor(jnp.concatenate([g1, g2], axis=0), params['pred'])
    pred_equ = _predictor(g_equ, params['pred'])
    pred_inv = _predictor(g_inv, params['pred'])
    oh1 = jax.nn.one_hot(label1, C); oh2 = jax.nn.one_hot(label2, C)
    mix = equ_lamb[:, None] * oh1 + (1.0 - equ_lamb)[:, None] * oh2
    logq = jax.nn.log_softmax(pred_equ, axis=-1)
    kl = jnp.where(mix > 0, mix * (jnp.log(jnp.clip(mix, 1e-12)) - logq), 0.0).sum() / G
    loss_reg = _reg(jnp.concatenate([nkn1, nkn2]), jnp.concatenate([nen1, nen2]), jnp.concatenate([rn1, rn2])) \
             + _reg(jnp.concatenate([ekn1, ekn2]), jnp.concatenate([een1, een2]), jnp.concatenate([re1, re2]))
    return pred_cau, pred_inv, loss_reg, kl


# speedup vs baseline: 2.1472x; 2.0239x over previous
import dataclasses
import functools

import jax
import jax.numpy as jnp
from jax import lax
from jax.experimental import pallas as pl
from jax.experimental.pallas import tpu as pltpu
from jax.experimental.pallas import tpu_sc as plsc

N = 10000; E = 320000; G = 128; IN = 128; D = 300; H = 600; C = 10
GAMMA = 0.4; INV_LAMB = 0.5

NSUB = 16
ROWS_MAIN = 632
BLK = 128


def _make_agg(n_edges, wc, weighted):
    total_blk = n_edges // BLK
    assert n_edges % BLK == 0
    nblk_lo = total_blk // NSUB
    extra = total_blk % NSUB
    mesh = plsc.VectorSubcoreMesh(core_axis_name="c", subcore_axis_name="s")

    rows_last = N - (NSUB - 1) * ROWS_MAIN

    out_type = [jax.ShapeDtypeStruct((N, wc), jnp.float32),
                jax.ShapeDtypeStruct((N, wc), jnp.float32)]
    scratch = [
        pltpu.VMEM_SHARED((N, wc), jnp.float32),
        pltpu.VMEM((BLK,), jnp.int32),
        pltpu.VMEM((BLK,), jnp.int32),
        pltpu.VMEM((BLK, wc), jnp.float32),
        pltpu.SemaphoreType.DMA,
    ]
    if weighted:
        scratch += [pltpu.VMEM((BLK,), jnp.float32)]

    def body(*refs):
        if weighted:
            (m_lo, m_hi, src, dst, wt, zr, o_lo, o_hi,
             acc, srcb, dstb, rowb, sem, wb) = refs
        else:
            (m_lo, m_hi, src, dst, zr, o_lo, o_hi,
             acc, srcb, dstb, rowb, sem) = refs
            wt = wb = None

        c = lax.axis_index("c")
        s = lax.axis_index("s")
        nblk = jnp.where(s < extra, nblk_lo + 1, nblk_lo)
        base = (s * nblk_lo + jnp.minimum(s, extra)) * BLK
        rbase = s * ROWS_MAIN

        @pl.when(s < NSUB - 1)
        def _():
            sl = pl.ds(rbase, ROWS_MAIN)
            pltpu.sync_copy(zr.at[sl], acc.at[sl])

        @pl.when(s == NSUB - 1)
        def _():
            sl = pl.ds((NSUB - 1) * ROWS_MAIN, rows_last)
            pltpu.sync_copy(zr.at[sl], acc.at[sl])

        plsc.subcore_barrier()

        def scale_rows():
            @pl.loop(0, BLK)
            def _(i):
                widx = jnp.full((16,), i, jnp.int32)
                wvec = plsc.load_gather(wb, [widx])
                for cc in range(wc // 16):
                    sl = pl.ds(cc * 16, 16)
                    rowb[i, sl] = rowb[i, sl] * wvec

        def do_block(mref, off):
            pltpu.sync_copy(src.at[pl.ds(off, BLK)], srcb)
            pltpu.sync_copy(dst.at[pl.ds(off, BLK)], dstb)
            if weighted:
                pltpu.sync_copy(wt.at[pl.ds(off, BLK)], wb)
            pltpu.async_copy(mref.at[srcb], rowb, sem).wait()
            if weighted:
                scale_rows()
            pltpu.sync_copy(rowb, acc.at[dstb], add=True)

        def main(mref):
            @pl.loop(0, nblk)
            def _(i):
                do_block(mref, base + i * BLK)

        @pl.when(c == 0)
        def _():
            main(m_lo)

        @pl.when(c == 1)
        def _():
            main(m_hi)

        plsc.subcore_barrier()

        def writeback(oref):
            @pl.when(s < NSUB - 1)
            def _():
                sl = pl.ds(rbase, ROWS_MAIN)
                pltpu.sync_copy(acc.at[sl], oref.at[sl])

            @pl.when(s == NSUB - 1)
            def _():
                sl = pl.ds((NSUB - 1) * ROWS_MAIN, rows_last)
                pltpu.sync_copy(acc.at[sl], oref.at[sl])

        @pl.when(c == 0)
        def _():
            writeback(o_lo)

        @pl.when(c == 1)
        def _():
            writeback(o_hi)

    cp = pltpu.CompilerParams(use_tc_tiling_on_sc=False)
    if "needs_layout_passes" in pltpu.CompilerParams.__dataclass_fields__:
        cp = dataclasses.replace(cp, needs_layout_passes=False)
    return pl.kernel(body, out_type=out_type, mesh=mesh,
                     scratch_types=scratch, compiler_params=cp)


_agg_e = _make_agg(E, 160, False)
_agg_e_w = _make_agg(E, 160, True)
_agg_2e = _make_agg(2 * E, 160, False)
_agg_2e_w = _make_agg(2 * E, 160, True)


def _sc_agg(m, src, dst, wt=None):
    mp = jnp.pad(m, ((0, 0), (0, 320 - D)))
    m_lo, m_hi = mp[:, :160], mp[:, 160:]
    n_edges = src.shape[0]
    zr = jnp.zeros((N, 160), jnp.float32)
    if wt is None:
        fn = _agg_e if n_edges == E else _agg_2e
        a_lo, a_hi = fn(m_lo, m_hi, src, dst, zr)
    else:
        fn = _agg_e_w if n_edges == E else _agg_2e_w
        a_lo, a_hi = fn(m_lo, m_hi, src, dst, wt, zr)
    return jnp.concatenate([a_lo, a_hi], axis=1)[:, :D]


def _matmul_bias_kernel(x_ref, w_ref, b_ref, o_ref):
    o_ref[...] = jnp.dot(x_ref[...], w_ref[...],
                         preferred_element_type=jnp.float32) + b_ref[...]


def _pallas_linear(x, w, b):
    m, k = x.shape
    k2, n = w.shape
    return pl.pallas_call(
        _matmul_bias_kernel,
        out_shape=jax.ShapeDtypeStruct((m, n), jnp.float32),
    )(x, w, b[None, :])


def _layer(x, W, b, src, dst, deg, node_mask=None, edge_mask=None):
    if node_mask is not None:
        x = x * node_mask
    m = _pallas_linear(x, W, b)
    agg = _sc_agg(m, src, dst, edge_mask)
    agg = agg / deg[:, None]
    return jax.nn.relu(agg + m)


def _encoder(x, ei, layers, deg, node_mask=None, edge_mask=None):
    src, dst = ei[0], ei[1]
    h = x
    for (W, b) in layers:
        h = _layer(h, W, b, src, dst, deg, node_mask, edge_mask)
    return h


def _causaler(x, ei, batch, deg, p):
    enc = _encoder(x, ei, p['enc'], deg)
    node_key = jax.nn.sigmoid(enc @ p['wn'] + p['bn'])
    src, dst = ei[0], ei[1]
    ef = jnp.concatenate([enc[src], enc[dst]], axis=1)
    edge_key = jax.nn.sigmoid(ef @ p['we'] + p['be'])
    nk = jax.ops.segment_sum(node_key[:, 0], batch, num_segments=G)
    nc = jax.ops.segment_sum(jnp.ones((x.shape[0],), x.dtype), batch, num_segments=G)
    eb = batch[src]
    ek = jax.ops.segment_sum(edge_key[:, 0], eb, num_segments=G)
    ec = jax.ops.segment_sum(jnp.ones((src.shape[0],), x.dtype), eb, num_segments=G)
    return node_key, edge_key, nk, nc - nk, ek, ec - ek, jnp.ones((G,), x.dtype), jnp.ones((G,), x.dtype)


def _pool(h, batch):
    s = jax.ops.segment_sum(h, batch, num_segments=G)
    c = jax.ops.segment_sum(jnp.ones((h.shape[0],), h.dtype), batch, num_segments=G)
    return s / jnp.clip(c, 1.0)[:, None]


def _predictor(h, p):
    z = _pallas_linear(h, p['w1'], p['b1'])
    mu = z.mean(0); var = z.var(0)
    z = (z - mu) / jnp.sqrt(var + 1e-5) * p['g'] + p['bt']
    z = jax.nn.relu(z)
    return _pallas_linear(z, p['w2'], p['b2'])


def _reg(cau, env, ratio):
    r = cau / (cau + env + 1e-8)
    return (jnp.abs(r - GAMMA) * ratio).mean()


def kernel(x1, edge_index1, x2, edge_index2, batch_idx, label1, label2, equ_lamb, params):
    src1, dst1 = edge_index1[0], edge_index1[1]
    src2, dst2 = edge_index2[0], edge_index2[1]
    ones_e = jnp.ones((E,), jnp.float32)
    deg1 = jnp.clip(jax.ops.segment_sum(ones_e, dst1, num_segments=N), 1.0)
    deg2 = jnp.clip(jax.ops.segment_sum(ones_e, dst2, num_segments=N), 1.0)
    deg_mix = jnp.clip(jax.ops.segment_sum(ones_e, dst1, num_segments=N)
                       + jax.ops.segment_sum(ones_e, dst2, num_segments=N), 1.0)

    xe1 = _encoder(x1, edge_index1, params['front'], deg1)
    xe2 = _encoder(x2, edge_index2, params['front'], deg2)
    nk1, ek1, nkn1, nen1, ekn1, een1, rn1, re1 = _causaler(x1, edge_index1, batch_idx, deg1, params['cau'])
    nk2, ek2, nkn2, nen2, ekn2, een2, rn2, re2 = _causaler(x2, edge_index2, batch_idx, deg2, params['cau'])
    h1 = _encoder(xe1, edge_index1, params['back'], deg1, node_mask=nk1, edge_mask=ek1[:, 0])
    h2 = _encoder(xe2, edge_index2, params['back'], deg2, node_mask=nk2, edge_mask=ek2[:, 0])
    lam_n = equ_lamb[batch_idx][:, None]
    x_equ = lam_n * xe1 + (1.0 - lam_n) * xe2
    x_inv = INV_LAMB * xe1 + (1.0 - INV_LAMB) * xe2
    ei_mix = jnp.concatenate([edge_index1, edge_index2], axis=1)
    l_e1 = equ_lamb[batch_idx[edge_index1[0]]]
    l_e2 = 1.0 - equ_lamb[batch_idx[edge_index2[0]]]
    w_equ = jnp.concatenate([l_e1, l_e2])
    w_inv = jnp.full((2 * E,), INV_LAMB, x1.dtype)
    h_equ = _encoder(x_equ, ei_mix, params['back'], deg_mix, edge_mask=w_equ)
    h_inv = _encoder(x_inv, ei_mix, params['back'], deg_mix, edge_mask=w_inv)
    g1 = _pool(h1, batch_idx); g2 = _pool(h2, batch_idx)
    g_equ = _pool(h_equ, batch_idx); g_inv = _pool(h_inv, batch_idx)
    pred_cau = _predictor(jnp.concatenate([g1, g2], axis=0), params['pred'])
    pred_equ = _predictor(g_equ, params['pred'])
    pred_inv = _predictor(g_inv, params['pred'])
    oh1 = jax.nn.one_hot(label1, C); oh2 = jax.nn.one_hot(label2, C)
    mix = equ_lamb[:, None] * oh1 + (1.0 - equ_lamb)[:, None] * oh2
    logq = jax.nn.log_softmax(pred_equ, axis=-1)
    kl = jnp.where(mix > 0, mix * (jnp.log(jnp.clip(mix, 1e-12)) - logq), 0.0).sum() / G
    loss_reg = _reg(jnp.concatenate([nkn1, nkn2]), jnp.concatenate([nen1, nen2]), jnp.concatenate([rn1, rn2])) \
             + _reg(jnp.concatenate([ekn1, ekn2]), jnp.concatenate([een1, een2]), jnp.concatenate([re1, re2]))
    return pred_cau, pred_inv, loss_reg, kl
